# baseline (speedup 1.0000x reference)
import functools

import jax
import jax.numpy as jnp
from jax import lax
from jax.experimental import pallas as pl
from jax.experimental.pallas import tpu as pltpu
from jax.experimental.pallas import tpu_sc as plsc

VOCAB = 1_000_000
EMB = 64
ROWS = 4096
COLS = 200
CHUNK = 128
TB = ROWS // CHUNK
S_ROWS = VOCAB // 2

_info = plsc.get_sparse_core_info()
NC = _info.num_cores
NS = _info.num_subcores
NW = NC * NS

SLABS = VOCAB // CHUNK

_mesh = plsc.VectorSubcoreMesh(core_axis_name="c", subcore_axis_name="s")

_params = pltpu.CompilerParams(use_tc_tiling_on_sc=True,
                               needs_layout_passes=False)


@functools.partial(
    pl.kernel,
    out_type=jax.ShapeDtypeStruct((S_ROWS, 2 * EMB), jnp.float32),
    mesh=_mesh,
    scratch_types=[
        pltpu.VMEM((EMB, CHUNK), jnp.float32),
        pltpu.VMEM((EMB, CHUNK), jnp.float32),
        pltpu.VMEM((EMB, CHUNK), jnp.float32),
        pltpu.VMEM((EMB, CHUNK), jnp.float32),
        pltpu.VMEM((EMB, EMB), jnp.float32),
        pltpu.VMEM((32, CHUNK), jnp.float32),
        pltpu.SemaphoreType.DMA((2,)),
        pltpu.SemaphoreType.DMA((2,)),
    ],
    compiler_params=_params,
)
def _detile_sc(wt_hbm, s_hbm, sl0, sl1, sb0, sb1, tsl, tsb, gsem, ssem):
    wid = lax.axis_index("s") * NC + lax.axis_index("c")
    slab = (sl0, sl1)
    sblock = (sb0, sb1)

    lane = lax.iota(jnp.int32, 16)
    evs = [qg * 16 + lane for qg in range(4)]

    def fire_gather(j, b):
        c0 = pl.multiple_of(j * CHUNK, CHUNK)
        pltpu.async_copy(wt_hbm.at[:, pl.ds(c0, CHUNK)], slab[b], gsem.at[b])

    def wait_gather(b):
        pltpu.make_async_copy(wt_hbm.at[:, pl.ds(0, CHUNK)], slab[b],
                              gsem.at[b]).wait()

    def fire_store(j, b):
        r0 = pl.multiple_of(j * EMB, EMB)
        pltpu.async_copy(sblock[b], s_hbm.at[pl.ds(r0, EMB)], ssem.at[b])

    def wait_store(b):
        pltpu.make_async_copy(sblock[b], s_hbm.at[pl.ds(0, EMB)],
                              ssem.at[b]).wait()

    def transpose(b):
        for pr in range(EMB):
            vals = [plsc.load_gather(
                        slab[b],
                        [evs[q % 4],
                         jnp.full((16,), 2 * pr + q // 4, jnp.int32)])
                    for q in range(8)]
            for q in range(8):
                sblock[b][pr, pl.ds(q * 16, 16)] = vals[q]

    def valid(j):
        return j < SLABS

    fire_gather(wid, 0)
    fire_gather(wid + NW, 1)

    n_pairs = 123

    def pair_body(ii, carry):
        i0 = ii * 2
        for b in range(2):
            i = i0 + b
            j = wid + i * NW

            @pl.when(valid(j))
            def _():
                wait_gather(b)

                @pl.when(ii > 0)
                def _():
                    wait_store(b)

                transpose(b)
                fire_store(j, b)

                jn = j + 2 * NW
                @pl.when(valid(jn))
                def _():
                    fire_gather(jn, b)

        return carry

    lax.fori_loop(0, n_pairs, pair_body, 0)
    wait_store(0)
    wait_store(1)

    @pl.when(jnp.equal(wid, 4))
    def _():
        pltpu.sync_copy(wt_hbm.at[:, pl.ds(VOCAB - EMB, EMB)], tsl)
        for pr in range(32):
            vals = [plsc.load_gather(
                        tsl,
                        [evs[q % 4],
                         jnp.full((16,), 2 * pr + q // 4, jnp.int32)])
                    for q in range(8)]
            for q in range(8):
                tsb[pr, pl.ds(q * 16, 16)] = vals[q]
        pltpu.sync_copy(tsb, s_hbm.at[pl.ds(S_ROWS - 32, 32)])


@functools.partial(
    pl.kernel,
    out_type=jax.ShapeDtypeStruct((COLS, 8, TB, 8, CHUNK), jnp.float32),
    mesh=_mesh,
    scratch_types=[
        pltpu.VMEM((COLS, CHUNK), jnp.int32),
        pltpu.VMEM((COLS, CHUNK), jnp.int32),
        pltpu.VMEM((CHUNK, 2 * EMB), jnp.float32),
        pltpu.VMEM((CHUNK, 2 * EMB), jnp.float32),
        pltpu.VMEM((8, 8, CHUNK), jnp.float32),
        pltpu.VMEM((8, 8, CHUNK), jnp.float32),
        pltpu.SemaphoreType.DMA((2,)),
        pltpu.SemaphoreType.DMA((2,)),
    ],
    compiler_params=_params,
)
def _embed_sc(idx_hbm, s_hbm, out_hbm, idx_v, idxh_v, fb0, fb1, tr0, tr1,
              gsem, ssem):
    wid = lax.axis_index("s") * NC + lax.axis_index("c")
    fb = (fb0, fb1)
    trans = (tr0, tr1)

    b0 = pl.multiple_of(wid * CHUNK, CHUNK)
    pltpu.sync_copy(idx_hbm.at[:, pl.ds(b0, CHUNK)], idx_v)

    def halve_row(r, carry):
        for g in range(8):
            idxh_v[r, pl.ds(g * 16, 16)] = (
                lax.shift_right_logical(idx_v[r, pl.ds(g * 16, 16)], 1))
        return carry

    lax.fori_loop(0, COLS, halve_row, 0)

    def fire_gather(t, b):
        pltpu.async_copy(s_hbm.at[idxh_v.at[t]], fb[b], gsem.at[b])

    def wait_gather(b):
        pltpu.make_async_copy(s_hbm.at[pl.ds(0, CHUNK)], fb[b],
                              gsem.at[b]).wait()

    def fire_store(t, b):
        pltpu.async_copy(trans[b], out_hbm.at[t, :, wid], ssem.at[b])

    def wait_store(b):
        pltpu.make_async_copy(trans[b], out_hbm.at[0, :, 0], ssem.at[b]).wait()

    lane = lax.iota(jnp.int32, 16)
    cvecs = [cg * 16 + lane for cg in range(8)]
    ones = jnp.full((16,), 1, jnp.int32)

    def transpose(t, b):
        par64 = []
        for cg in range(8):
            iv = idx_v[t, pl.ds(cg * 16, 16)]
            par64.append(lax.shift_left(lax.bitwise_and(iv, ones), 6))
        for e in range(EMB):
            xs = [par64[cg] + e for cg in range(8)]
            vals = [plsc.load_gather(fb[b], [cvecs[cg], xs[cg]])
                    for cg in range(8)]
            for cg in range(8):
                trans[b][e // 8, e % 8, pl.ds(cg * 16, 16)] = vals[cg]

    fire_gather(0, 0)
    fire_gather(1, 1)

    n_pairs = COLS // 2

    def pair_body(tt, carry):
        t0 = tt * 2
        for b in range(2):
            t = t0 + b
            wait_gather(b)

            @pl.when(tt > 0)
            def _():
                wait_store(b)

            transpose(t, b)
            fire_store(t, b)

            @pl.when(tt < n_pairs - 1)
            def _():
                fire_gather(t + 2, b)

        return carry

    lax.fori_loop(0, n_pairs, pair_body, 0)
    wait_store(0)
    wait_store(1)


def kernel(input_tensor, weight):
    wt = weight.swapaxes(0, 1)
    s = _detile_sc(wt)
    idx_t = input_tensor.astype(jnp.int32).swapaxes(0, 1)
    out5 = _embed_sc(idx_t, s)
    return out5.transpose(2, 4, 0, 1, 3).reshape(ROWS, COLS, EMB)


# speedup vs baseline: 1.3521x; 1.3521x over previous
import functools

import jax
import jax.numpy as jnp
from jax import lax
from jax.experimental import pallas as pl
from jax.experimental.pallas import tpu as pltpu
from jax.experimental.pallas import tpu_sc as plsc

VOCAB = 1_000_000
EMB = 64
ROWS = 4096
COLS = 200
CHUNK = 128
TB = ROWS // CHUNK

_info = plsc.get_sparse_core_info()
NC = _info.num_cores
NS = _info.num_subcores
NW = NC * NS

_mesh = plsc.VectorSubcoreMesh(core_axis_name="c", subcore_axis_name="s")


@functools.partial(
    pl.kernel,
    out_type=jax.ShapeDtypeStruct((COLS, 8, TB, 8, CHUNK), jnp.float32),
    mesh=_mesh,
    scratch_types=[
        pltpu.VMEM((COLS, CHUNK), jnp.int32),
        pltpu.VMEM((CHUNK, EMB), jnp.float32),
        pltpu.VMEM((CHUNK, EMB), jnp.float32),
        pltpu.VMEM((8, 8, CHUNK), jnp.float32),
        pltpu.VMEM((8, 8, CHUNK), jnp.float32),
        pltpu.SemaphoreType.DMA((2,)),
        pltpu.SemaphoreType.DMA((2,)),
    ],
    compiler_params=pltpu.CompilerParams(use_tc_tiling_on_sc=False,
                                         needs_layout_passes=False),
)
def _embed_sc(idx_hbm, table_hbm, out_hbm, idx_v, rows0, rows1, tr0, tr1,
              gsem, ssem):
    wid = lax.axis_index("s") * NC + lax.axis_index("c")
    rows = (rows0, rows1)
    trans = (tr0, tr1)

    b0 = pl.multiple_of(wid * CHUNK, CHUNK)
    pltpu.sync_copy(idx_hbm.at[:, pl.ds(b0, CHUNK)], idx_v)

    def fire_gather(t, b):
        pltpu.async_copy(table_hbm.at[idx_v.at[t]], rows[b], gsem.at[b])

    def wait_gather(b):
        pltpu.make_async_copy(table_hbm.at[pl.ds(0, CHUNK)], rows[b],
                              gsem.at[b]).wait()

    def fire_store(t, b):
        pltpu.async_copy(trans[b], out_hbm.at[t, :, wid], ssem.at[b])

    def wait_store(b):
        pltpu.make_async_copy(trans[b], out_hbm.at[0, :, 0], ssem.at[b]).wait()

    lane = lax.iota(jnp.int32, 16)
    cvecs = [cg * 16 + lane for cg in range(8)]

    def transpose(b):
        for e in range(EMB):
            evec = jnp.full((16,), e, jnp.int32)
            vals = [plsc.load_gather(rows[b], [cvecs[cg], evec])
                    for cg in range(8)]
            for cg in range(8):
                trans[b][e // 8, e % 8, pl.ds(cg * 16, 16)] = vals[cg]

    fire_gather(0, 0)
    fire_gather(1, 1)

    n_pairs = COLS // 2

    def pair_body(tt, carry):
        t0 = tt * 2
        for b in range(2):
            t = t0 + b
            wait_gather(b)

            @pl.when(tt > 0)
            def _():
                wait_store(b)

            transpose(b)
            fire_store(t, b)

            @pl.when(tt < n_pairs - 1)
            def _():
                fire_gather(t + 2, b)

        return carry

    lax.fori_loop(0, n_pairs, pair_body, 0)
    wait_store(0)
    wait_store(1)


def kernel(input_tensor, weight):
    idx_t = input_tensor.astype(jnp.int32).swapaxes(0, 1)
    out5 = _embed_sc(idx_t, weight)
    return out5.transpose(2, 4, 0, 1, 3).reshape(ROWS, COLS, EMB)
